# Initial kernel scaffold; baseline (speedup 1.0000x reference)
import jax
import jax.numpy as jnp
from jax.experimental import pallas as pl


def kernel(x, edge_index, W1_self, W1_neigh, b1, W2_self, W2_neigh, b2):
    raise NotImplementedError("write your pallas kernel here")


# speedup vs baseline: 12.6028x; 12.6028x over previous
import functools

import jax
import jax.numpy as jnp
from jax import lax
from jax.experimental import pallas as pl
from jax.experimental.pallas import tpu as pltpu
from jax.experimental.pallas import tpu_sc as plsc

NC = 2
NS = 16
NW = NC * NS
EB = 125


def _acc_rows(n_nodes):
    rpt = (-(-n_nodes // NS) + 7) // 8 * 8
    return rpt, NS * rpt


def _make_sc_agg(n_nodes, d, n_edges, with_deg):
    k_steps = n_edges // (NW * EB)
    rpt, n_pad = _acc_rows(n_nodes)
    mesh = plsc.VectorSubcoreMesh(core_axis_name="c", subcore_axis_name="s")

    outs = [jax.ShapeDtypeStruct((NC, n_pad, d), jnp.float32)]
    scratch = [
        pltpu.VMEM((k_steps, EB), jnp.int32),
        pltpu.VMEM((k_steps, EB), jnp.int32),
        pltpu.VMEM((EB, d), jnp.float32),
        pltpu.VMEM_SHARED((n_pad, d), jnp.float32),
        pltpu.SemaphoreType.DMA,
    ]
    if with_deg:
        outs.append(jax.ShapeDtypeStruct((NC, n_pad, d), jnp.float32))
        scratch += [
            pltpu.VMEM((EB, d), jnp.float32),
            pltpu.VMEM_SHARED((n_pad, d), jnp.float32),
        ]

    def body(*refs):
        if with_deg:
            (p_hbm, src_hbm, dst_hbm, zeros_hbm, ones_hbm,
             acc_out, deg_out,
             src_v, dst_v, msg_v, acc_sh, sem, ones_v, deg_sh) = refs
        else:
            (p_hbm, src_hbm, dst_hbm, zeros_hbm,
             acc_out,
             src_v, dst_v, msg_v, acc_sh, sem) = refs

        c = lax.axis_index("c")
        s = lax.axis_index("s")
        w = c * NS + s
        row0 = s * rpt

        pltpu.sync_copy(zeros_hbm, acc_sh.at[pl.ds(row0, rpt)])
        if with_deg:
            pltpu.sync_copy(zeros_hbm, deg_sh.at[pl.ds(row0, rpt)])
            pltpu.sync_copy(ones_hbm, ones_v)
        pltpu.sync_copy(src_hbm.at[pl.ds(w * k_steps, k_steps)], src_v)
        pltpu.sync_copy(dst_hbm.at[pl.ds(w * k_steps, k_steps)], dst_v)
        plsc.subcore_barrier()

        def step(k, carry):
            pltpu.async_copy(p_hbm.at[src_v.at[k]], msg_v, sem).wait()
            pltpu.sync_copy(msg_v, acc_sh.at[dst_v.at[k]], add=True)
            if with_deg:
                pltpu.sync_copy(ones_v, deg_sh.at[dst_v.at[k]], add=True)
            return carry

        lax.fori_loop(0, k_steps, step, 0)
        plsc.subcore_barrier()

        pltpu.sync_copy(acc_sh.at[pl.ds(row0, rpt)],
                        acc_out.at[c, pl.ds(row0, rpt)])
        if with_deg:
            pltpu.sync_copy(deg_sh.at[pl.ds(row0, rpt)],
                            deg_out.at[c, pl.ds(row0, rpt)])

    return pl.kernel(
        body, out_type=outs, mesh=mesh, scratch_types=scratch,
        compiler_params=pltpu.CompilerParams(use_tc_tiling_on_sc=False))


def _proj2(x, w_self, w_neigh):
    n, din = x.shape
    d = w_self.shape[1]
    blk = 2000

    def body(x_ref, ws_ref, wn_ref, s_ref, p_ref):
        xb = x_ref[...]
        s_ref[...] = jnp.dot(xb, ws_ref[...], preferred_element_type=jnp.float32)
        p_ref[...] = jnp.dot(xb, wn_ref[...], preferred_element_type=jnp.float32)

    return pl.pallas_call(
        body,
        grid=(n // blk,),
        in_specs=[
            pl.BlockSpec((blk, din), lambda i: (i, 0)),
            pl.BlockSpec((din, d), lambda i: (0, 0)),
            pl.BlockSpec((din, d), lambda i: (0, 0)),
        ],
        out_specs=[
            pl.BlockSpec((blk, d), lambda i: (i, 0)),
            pl.BlockSpec((blk, d), lambda i: (i, 0)),
        ],
        out_shape=[jax.ShapeDtypeStruct((n, d), jnp.float32)] * 2,
    )(x, w_self, w_neigh)


def _combine1(s1, acc, deg, b1):
    n, d = s1.shape
    blk = 2000

    def body(s_ref, a_ref, g_ref, b_ref, h_ref, r_ref):
        agg = a_ref[0] + a_ref[1]
        dg = g_ref[0] + g_ref[1]
        r = 1.0 / jnp.maximum(dg, 1.0)
        h_ref[...] = jnp.maximum(s_ref[...] + agg * r + b_ref[...], 0.0)
        r_ref[...] = r

    return pl.pallas_call(
        body,
        grid=(n // blk,),
        in_specs=[
            pl.BlockSpec((blk, d), lambda i: (i, 0)),
            pl.BlockSpec((NC, blk, d), lambda i: (0, i, 0)),
            pl.BlockSpec((NC, blk, d), lambda i: (0, i, 0)),
            pl.BlockSpec((1, d), lambda i: (0, 0)),
        ],
        out_specs=[
            pl.BlockSpec((blk, d), lambda i: (i, 0)),
            pl.BlockSpec((blk, d), lambda i: (i, 0)),
        ],
        out_shape=[jax.ShapeDtypeStruct((n, d), jnp.float32)] * 2,
    )(s1, acc, deg, b1)


def _out_layer(h, acc2, rdeg, w2s, w2n, b2):
    n, d = h.shape
    dout = w2s.shape[1]
    blk = 2000

    def body(h_ref, a_ref, r_ref, ws_ref, wn_ref, b_ref, o_ref):
        m = (a_ref[0] + a_ref[1]) * r_ref[...]
        o_ref[...] = (
            jnp.dot(h_ref[...], ws_ref[...], preferred_element_type=jnp.float32)
            + jnp.dot(m, wn_ref[...], preferred_element_type=jnp.float32)
            + b_ref[...]
        )

    return pl.pallas_call(
        body,
        grid=(n // blk,),
        in_specs=[
            pl.BlockSpec((blk, d), lambda i: (i, 0)),
            pl.BlockSpec((NC, blk, d), lambda i: (0, i, 0)),
            pl.BlockSpec((blk, d), lambda i: (i, 0)),
            pl.BlockSpec((d, dout), lambda i: (0, 0)),
            pl.BlockSpec((d, dout), lambda i: (0, 0)),
            pl.BlockSpec((1, dout), lambda i: (0, 0)),
        ],
        out_specs=pl.BlockSpec((blk, dout), lambda i: (i, 0)),
        out_shape=jax.ShapeDtypeStruct((n, dout), jnp.float32),
    )(h, acc2, rdeg, w2s, w2n, b2)


def kernel(x, edge_index, W1_self, W1_neigh, b1, W2_self, W2_neigh, b2):
    n, _ = x.shape
    d = W1_self.shape[1]
    e = edge_index.shape[1]
    k_rows = e // EB

    src = edge_index[0].astype(jnp.int32).reshape(k_rows, EB)
    dst = edge_index[1].astype(jnp.int32).reshape(k_rows, EB)
    rpt, _ = _acc_rows(n)
    zeros = jnp.zeros((rpt, d), jnp.float32)
    ones = jnp.ones((EB, d), jnp.float32)

    s1, p1 = _proj2(x, W1_self, W1_neigh)
    agg1, deg = _make_sc_agg(n, d, e, True)(p1, src, dst, zeros, ones)
    h, rdeg = _combine1(s1, agg1, deg, b1.reshape(1, d))
    (agg2,) = _make_sc_agg(n, d, e, False)(h, src, dst, zeros)
    return _out_layer(h, agg2, rdeg, W2_self, W2_neigh,
                      b2.reshape(1, W2_self.shape[1]))


# speedup vs baseline: 19.8837x; 1.5777x over previous
import functools

import jax
import jax.numpy as jnp
from jax import lax
from jax.experimental import pallas as pl
from jax.experimental.pallas import tpu as pltpu
from jax.experimental.pallas import tpu_sc as plsc

NC = 2
NS = 16
NW = NC * NS
EB = 125


def _acc_rows(n_nodes):
    rpt = (-(-n_nodes // NS) + 7) // 8 * 8
    return rpt, NS * rpt


NB = 4


def _make_sc_agg(n_nodes, d, n_edges, with_deg):
    k_steps = n_edges // (NW * EB)
    n_groups = k_steps // NB
    rpt, n_pad = _acc_rows(n_nodes)
    mesh = plsc.VectorSubcoreMesh(core_axis_name="c", subcore_axis_name="s")

    outs = [jax.ShapeDtypeStruct((NC, n_pad, d), jnp.float32)]
    scratch = [
        pltpu.VMEM((k_steps, EB), jnp.int32),
        pltpu.VMEM((k_steps, EB), jnp.int32),
        pltpu.VMEM((2 * NB, EB, d), jnp.float32),
        pltpu.VMEM_SHARED((n_pad, d), jnp.float32),
        [pltpu.SemaphoreType.DMA] * 2,
        [pltpu.SemaphoreType.DMA] * 2,
    ]
    if with_deg:
        outs.append(jax.ShapeDtypeStruct((NC, n_pad, d), jnp.float32))
        scratch += [
            pltpu.VMEM((EB, d), jnp.float32),
            pltpu.VMEM_SHARED((n_pad, d), jnp.float32),
        ]

    def body(*refs):
        if with_deg:
            (p_hbm, src_hbm, dst_hbm, zeros_hbm, ones_hbm,
             acc_out, deg_out,
             src_v, dst_v, msg_v, acc_sh, sem_g, sem_s, ones_v, deg_sh) = refs
        else:
            (p_hbm, src_hbm, dst_hbm, zeros_hbm,
             acc_out,
             src_v, dst_v, msg_v, acc_sh, sem_g, sem_s) = refs

        c = lax.axis_index("c")
        s = lax.axis_index("s")
        w = c * NS + s
        row0 = s * rpt

        pltpu.sync_copy(zeros_hbm, acc_sh.at[pl.ds(row0, rpt)])
        if with_deg:
            pltpu.sync_copy(zeros_hbm, deg_sh.at[pl.ds(row0, rpt)])
            pltpu.sync_copy(ones_hbm, ones_v)
        pltpu.sync_copy(src_hbm.at[pl.ds(w * k_steps, k_steps)], src_v)
        pltpu.sync_copy(dst_hbm.at[pl.ds(w * k_steps, k_steps)], dst_v)
        plsc.subcore_barrier()

        def fire_gathers(g, half):
            for b in range(NB):
                pltpu.async_copy(p_hbm.at[src_v.at[g * NB + b]],
                                 msg_v.at[half * NB + b], sem_g[half])

        def drain(sem, count):
            for _ in range(count):
                pltpu.make_async_copy(p_hbm.at[pl.ds(0, EB)],
                                      msg_v.at[0], sem).wait()

        def fire_scatters(g, half):
            for b in range(NB):
                pltpu.async_copy(msg_v.at[half * NB + b],
                                 acc_sh.at[dst_v.at[g * NB + b]],
                                 sem_s[half], add=True)
                if with_deg:
                    pltpu.async_copy(ones_v,
                                     deg_sh.at[dst_v.at[g * NB + b]],
                                     sem_s[half], add=True)

        fire_gathers(0, 0)

        def group_steps(g, half):
            drain(sem_g[half], NB)
            fire_scatters(g, half)

            @pl.when(g + 1 < n_groups)
            def _():
                fire_gathers(g + 1, 1 - half)

            drain(sem_s[half], NB * (2 if with_deg else 1))

        def body2(j2, carry):
            group_steps(j2 * 2, 0)
            group_steps(j2 * 2 + 1, 1)
            return carry

        lax.fori_loop(0, n_groups // 2, body2, 0)
        plsc.subcore_barrier()

        pltpu.sync_copy(acc_sh.at[pl.ds(row0, rpt)],
                        acc_out.at[c, pl.ds(row0, rpt)])
        if with_deg:
            pltpu.sync_copy(deg_sh.at[pl.ds(row0, rpt)],
                            deg_out.at[c, pl.ds(row0, rpt)])

    return pl.kernel(
        body, out_type=outs, mesh=mesh, scratch_types=scratch,
        compiler_params=pltpu.CompilerParams(use_tc_tiling_on_sc=False))


def _proj2(x, w_self, w_neigh):
    n, din = x.shape
    d = w_self.shape[1]
    blk = 2000

    def body(x_ref, ws_ref, wn_ref, s_ref, p_ref):
        xb = x_ref[...]
        s_ref[...] = jnp.dot(xb, ws_ref[...], preferred_element_type=jnp.float32)
        p_ref[...] = jnp.dot(xb, wn_ref[...], preferred_element_type=jnp.float32)

    return pl.pallas_call(
        body,
        grid=(n // blk,),
        in_specs=[
            pl.BlockSpec((blk, din), lambda i: (i, 0)),
            pl.BlockSpec((din, d), lambda i: (0, 0)),
            pl.BlockSpec((din, d), lambda i: (0, 0)),
        ],
        out_specs=[
            pl.BlockSpec((blk, d), lambda i: (i, 0)),
            pl.BlockSpec((blk, d), lambda i: (i, 0)),
        ],
        out_shape=[jax.ShapeDtypeStruct((n, d), jnp.float32)] * 2,
    )(x, w_self, w_neigh)


def _combine1(s1, acc, deg, b1):
    n, d = s1.shape
    blk = 2000

    def body(s_ref, a_ref, g_ref, b_ref, h_ref, r_ref):
        agg = a_ref[0] + a_ref[1]
        dg = g_ref[0] + g_ref[1]
        r = 1.0 / jnp.maximum(dg, 1.0)
        h_ref[...] = jnp.maximum(s_ref[...] + agg * r + b_ref[...], 0.0)
        r_ref[...] = r

    return pl.pallas_call(
        body,
        grid=(n // blk,),
        in_specs=[
            pl.BlockSpec((blk, d), lambda i: (i, 0)),
            pl.BlockSpec((NC, blk, d), lambda i: (0, i, 0)),
            pl.BlockSpec((NC, blk, d), lambda i: (0, i, 0)),
            pl.BlockSpec((1, d), lambda i: (0, 0)),
        ],
        out_specs=[
            pl.BlockSpec((blk, d), lambda i: (i, 0)),
            pl.BlockSpec((blk, d), lambda i: (i, 0)),
        ],
        out_shape=[jax.ShapeDtypeStruct((n, d), jnp.float32)] * 2,
    )(s1, acc, deg, b1)


def _out_layer(h, acc2, rdeg, w2s, w2n, b2):
    n, d = h.shape
    dout = w2s.shape[1]
    blk = 2000

    def body(h_ref, a_ref, r_ref, ws_ref, wn_ref, b_ref, o_ref):
        m = (a_ref[0] + a_ref[1]) * r_ref[...]
        o_ref[...] = (
            jnp.dot(h_ref[...], ws_ref[...], preferred_element_type=jnp.float32)
            + jnp.dot(m, wn_ref[...], preferred_element_type=jnp.float32)
            + b_ref[...]
        )

    return pl.pallas_call(
        body,
        grid=(n // blk,),
        in_specs=[
            pl.BlockSpec((blk, d), lambda i: (i, 0)),
            pl.BlockSpec((NC, blk, d), lambda i: (0, i, 0)),
            pl.BlockSpec((blk, d), lambda i: (i, 0)),
            pl.BlockSpec((d, dout), lambda i: (0, 0)),
            pl.BlockSpec((d, dout), lambda i: (0, 0)),
            pl.BlockSpec((1, dout), lambda i: (0, 0)),
        ],
        out_specs=pl.BlockSpec((blk, dout), lambda i: (i, 0)),
        out_shape=jax.ShapeDtypeStruct((n, dout), jnp.float32),
    )(h, acc2, rdeg, w2s, w2n, b2)


def kernel(x, edge_index, W1_self, W1_neigh, b1, W2_self, W2_neigh, b2):
    n, _ = x.shape
    d = W1_self.shape[1]
    e = edge_index.shape[1]
    k_rows = e // EB

    src = edge_index[0].astype(jnp.int32).reshape(k_rows, EB)
    dst = edge_index[1].astype(jnp.int32).reshape(k_rows, EB)
    rpt, _ = _acc_rows(n)
    zeros = jnp.zeros((rpt, d), jnp.float32)
    ones = jnp.ones((EB, d), jnp.float32)

    s1, p1 = _proj2(x, W1_self, W1_neigh)
    agg1, deg = _make_sc_agg(n, d, e, True)(p1, src, dst, zeros, ones)
    h, rdeg = _combine1(s1, agg1, deg, b1.reshape(1, d))
    (agg2,) = _make_sc_agg(n, d, e, False)(h, src, dst, zeros)
    return _out_layer(h, agg2, rdeg, W2_self, W2_neigh,
                      b2.reshape(1, W2_self.shape[1]))
